# baseline (speedup 1.0000x reference)
import functools

import jax
import jax.numpy as jnp
from jax import lax
from jax.experimental import pallas as pl
from jax.experimental.pallas import tpu as pltpu
from jax.experimental.pallas import tpu_sc as plsc

N = 10000
E = 320000
D_IN = 26
H = 128
OUT = 64
EPS = 1e-5

NC = 2
NS = 16
NW = NC * NS
EPW = E // NW
K = 128
EPAD = NW * 80 * K
NCHUNK = EPAD // (NW * K)
NPAD = 10240
RPT = NPAD // NS
TRASH = NPAD - 8

B = 1000
GRID = N // B


@functools.cache
def _sc_kernels():
    mesh = plsc.VectorSubcoreMesh(core_axis_name="c", subcore_axis_name="s",
                                  num_cores=NC, num_subcores=NS)

    @functools.partial(
        pl.kernel,
        out_type=jax.ShapeDtypeStruct((NC, NPAD, H), jnp.float32),
        mesh=mesh,
        scratch_types=[
            pltpu.VMEM((NCHUNK, K), jnp.int32),
            pltpu.VMEM((K, H), jnp.float32),
            pltpu.VMEM_SHARED((NPAD, H), jnp.float32),
        ],
    )
    def _sc_deg(cols_hbm, zeros_hbm, ones_hbm, degp_hbm, cidx_v, ones_v, deg_sh):
        c = lax.axis_index("c")
        s = lax.axis_index("s")
        tid = c * NS + s
        pltpu.sync_copy(cols_hbm.at[tid], cidx_v)
        pltpu.sync_copy(ones_hbm, ones_v)
        pltpu.sync_copy(zeros_hbm.at[pl.ds(s * RPT, RPT)],
                        deg_sh.at[pl.ds(s * RPT, RPT)])
        plsc.subcore_barrier()

        def body(j, carry):
            pltpu.sync_copy(ones_v, deg_sh.at[cidx_v.at[j]], add=True)
            return carry

        lax.fori_loop(0, NCHUNK, body, 0)
        plsc.subcore_barrier()
        pltpu.sync_copy(deg_sh.at[pl.ds(s * RPT, RPT)],
                        degp_hbm.at[c, pl.ds(s * RPT, RPT)])

    @functools.partial(
        pl.kernel,
        out_type=jax.ShapeDtypeStruct((NC, NPAD, H), jnp.float32),
        mesh=mesh,
        scratch_types=[
            pltpu.VMEM((2, 2, K), jnp.int32),
            pltpu.VMEM((2, 2, K), jnp.int32),
            pltpu.VMEM((K, H), jnp.float32),
            pltpu.VMEM((K, H), jnp.float32),
            pltpu.VMEM_SHARED((NPAD, H), jnp.float32),
            pltpu.SemaphoreType.DMA,
            pltpu.SemaphoreType.DMA,
            pltpu.SemaphoreType.DMA,
            pltpu.SemaphoreType.DMA,
        ],
    )
    def _sc_conv(rc_hbm, t_hbm, zeros_hbm, sp_hbm,
                 ia_v, ib_v, gbuf0_v, gbuf1_v, acc_sh, sem0, sem1, sema,
                 semb):
        c = lax.axis_index("c")
        s = lax.axis_index("s")
        tid = c * NS + s
        pltpu.sync_copy(zeros_hbm.at[pl.ds(s * RPT, RPT)],
                        acc_sh.at[pl.ds(s * RPT, RPT)])
        plsc.subcore_barrier()

        def g_start(ibuf, row, gb, sem):
            pltpu.async_copy(t_hbm.at[ibuf.at[row, 0]], gb, sem)

        def g_wait(ibuf, row, gb, sem):
            pltpu.make_async_copy(t_hbm.at[ibuf.at[row, 0]], gb, sem).wait()

        def scat(ibuf, row, gb):
            pltpu.sync_copy(gb, acc_sh.at[ibuf.at[row, 1]], add=True)

        def i_start(j2, ibuf, sem):
            pltpu.async_copy(rc_hbm.at[tid, pl.ds(j2, 2)], ibuf, sem)

        def i_wait(j2, ibuf, sem):
            pltpu.make_async_copy(rc_hbm.at[tid, pl.ds(j2, 2)], ibuf,
                                  sem).wait()

        NQUAD = NCHUNK // 4
        pltpu.sync_copy(rc_hbm.at[tid, pl.ds(0, 2)], ia_v)
        i_start(2, ib_v, semb)
        g_start(ia_v, 0, gbuf0_v, sem0)

        def body(i, carry):
            q = 4 * i
            g_start(ia_v, 1, gbuf1_v, sem1)
            g_wait(ia_v, 0, gbuf0_v, sem0)
            scat(ia_v, 0, gbuf0_v)
            i_wait(q + 2, ib_v, semb)
            g_start(ib_v, 0, gbuf0_v, sem0)
            g_wait(ia_v, 1, gbuf1_v, sem1)
            scat(ia_v, 1, gbuf1_v)

            @pl.when(q + 5 < NCHUNK)
            def _():
                i_start(q + 4, ia_v, sema)

            g_start(ib_v, 1, gbuf1_v, sem1)
            g_wait(ib_v, 0, gbuf0_v, sem0)
            scat(ib_v, 0, gbuf0_v)

            @pl.when(q + 5 < NCHUNK)
            def _():
                i_wait(q + 4, ia_v, sema)
                g_start(ia_v, 0, gbuf0_v, sem0)

            g_wait(ib_v, 1, gbuf1_v, sem1)
            scat(ib_v, 1, gbuf1_v)

            @pl.when(q + 7 < NCHUNK)
            def _():
                i_start(q + 6, ib_v, semb)

            return carry

        lax.fori_loop(0, NQUAD, body, 0)
        plsc.subcore_barrier()
        pltpu.sync_copy(acc_sh.at[pl.ds(s * RPT, RPT)],
                        sp_hbm.at[c, pl.ds(s * RPT, RPT)])

    return _sc_deg, _sc_conv


def _dinv_block(degp):
    deg = degp[0, :, 0:1] + degp[1, :, 0:1] + 1.0
    return lax.rsqrt(deg)


def _prologue_body(x_ref, w0_ref, b0_ref, w1_ref, b1_ref, wg1_ref, degp_ref,
                   gi_ref, t1_ref, dinv16_ref):
    h = jnp.maximum(
        jnp.dot(x_ref[...], w0_ref[...], preferred_element_type=jnp.float32)
        + b0_ref[...], 0.0)
    gi = jnp.dot(h, w1_ref[...], preferred_element_type=jnp.float32) + b1_ref[...]
    gi_ref[...] = gi
    dinv = _dinv_block(degp_ref[...])
    dinv16_ref[...] = dinv * jnp.ones((1, 16), jnp.float32)
    t1_ref[...] = jnp.dot(gi, wg1_ref[...],
                          preferred_element_type=jnp.float32) * dinv


def _prologue(x, W0, b0, W1, b1, Wg1, degp):
    return pl.pallas_call(
        _prologue_body,
        grid=(GRID,),
        in_specs=[
            pl.BlockSpec((B, D_IN), lambda i: (i, 0)),
            pl.BlockSpec((D_IN, H), lambda i: (0, 0)),
            pl.BlockSpec((1, H), lambda i: (0, 0)),
            pl.BlockSpec((H, H), lambda i: (0, 0)),
            pl.BlockSpec((1, H), lambda i: (0, 0)),
            pl.BlockSpec((H, H), lambda i: (0, 0)),
            pl.BlockSpec((NC, B, H), lambda i: (0, i, 0)),
        ],
        out_specs=[
            pl.BlockSpec((B, H), lambda i: (i, 0)),
            pl.BlockSpec((B, H), lambda i: (i, 0)),
            pl.BlockSpec((B, 16), lambda i: (i, 0)),
        ],
        out_shape=[
            jax.ShapeDtypeStruct((N, H), jnp.float32),
            jax.ShapeDtypeStruct((N, H), jnp.float32),
            jax.ShapeDtypeStruct((N, 16), jnp.float32),
        ],
    )(x, W0, b0, W1, b1, Wg1, degp)


def _combine_body(sp_ref, t_ref, dinv16_ref, bg_ref, g_ref, st_ref):
    i = pl.program_id(0)
    sp = sp_ref[...]
    dinv = dinv16_ref[...][:, 0:1]
    g = (sp[0] + sp[1] + t_ref[...]) * dinv + bg_ref[...]
    g_ref[...] = g

    @pl.when(i == 0)
    def _():
        st_ref[...] = jnp.zeros((8, H), jnp.float32)

    st_ref[0:1, :] += jnp.sum(g, axis=0, keepdims=True)
    st_ref[1:2, :] += jnp.sum(g * g, axis=0, keepdims=True)


def _combine(sp, t, dinv16, bg):
    return pl.pallas_call(
        _combine_body,
        grid=(GRID,),
        in_specs=[
            pl.BlockSpec((NC, B, H), lambda i: (0, i, 0)),
            pl.BlockSpec((B, H), lambda i: (i, 0)),
            pl.BlockSpec((B, 16), lambda i: (i, 0)),
            pl.BlockSpec((1, H), lambda i: (0, 0)),
        ],
        out_specs=[
            pl.BlockSpec((B, H), lambda i: (i, 0)),
            pl.BlockSpec((8, H), lambda i: (0, 0)),
        ],
        out_shape=[
            jax.ShapeDtypeStruct((N, H), jnp.float32),
            jax.ShapeDtypeStruct((8, H), jnp.float32),
        ],
    )(sp, t, dinv16, bg)


def _mid_body(g_ref, st_ref, gamma_ref, beta_ref, gi_ref, wg2_ref, dinv16_ref,
              t2_ref):
    mu = st_ref[0:1, :] / N
    var = st_ref[1:2, :] / N - mu * mu
    inv = lax.rsqrt(var + EPS)
    xh = (g_ref[...] - mu) * (gamma_ref[...] * inv) + beta_ref[...]
    r = jnp.maximum(xh, 0.0) + gi_ref[...]
    dinv = dinv16_ref[...][:, 0:1]
    t2_ref[...] = jnp.dot(r, wg2_ref[...],
                          preferred_element_type=jnp.float32) * dinv


def _mid(g1, st1, gamma1, beta1, gi, Wg2, dinv16):
    return pl.pallas_call(
        _mid_body,
        grid=(GRID,),
        in_specs=[
            pl.BlockSpec((B, H), lambda i: (i, 0)),
            pl.BlockSpec((8, H), lambda i: (0, 0)),
            pl.BlockSpec((1, H), lambda i: (0, 0)),
            pl.BlockSpec((1, H), lambda i: (0, 0)),
            pl.BlockSpec((B, H), lambda i: (i, 0)),
            pl.BlockSpec((H, H), lambda i: (0, 0)),
            pl.BlockSpec((B, 16), lambda i: (i, 0)),
        ],
        out_specs=pl.BlockSpec((B, H), lambda i: (i, 0)),
        out_shape=jax.ShapeDtypeStruct((N, H), jnp.float32),
    )(g1, st1, gamma1, beta1, gi, Wg2, dinv16)


def _epilogue_body(g_ref, st_ref, gamma_ref, beta_ref, wo_ref, bo_ref, o_ref):
    mu = st_ref[0:1, :] / N
    var = st_ref[1:2, :] / N - mu * mu
    inv = lax.rsqrt(var + EPS)
    xh = (g_ref[...] - mu) * (gamma_ref[...] * inv) + beta_ref[...]
    o_ref[...] = jnp.dot(xh, wo_ref[...],
                         preferred_element_type=jnp.float32) + bo_ref[...]


def _epilogue(g2, st2, gamma2, beta2, Wo, bo):
    return pl.pallas_call(
        _epilogue_body,
        grid=(GRID,),
        in_specs=[
            pl.BlockSpec((B, H), lambda i: (i, 0)),
            pl.BlockSpec((8, H), lambda i: (0, 0)),
            pl.BlockSpec((1, H), lambda i: (0, 0)),
            pl.BlockSpec((1, H), lambda i: (0, 0)),
            pl.BlockSpec((H, OUT), lambda i: (0, 0)),
            pl.BlockSpec((1, OUT), lambda i: (0, 0)),
        ],
        out_specs=pl.BlockSpec((B, OUT), lambda i: (i, 0)),
        out_shape=jax.ShapeDtypeStruct((N, OUT), jnp.float32),
    )(g2, st2, gamma2, beta2, Wo, bo)


def kernel(x, edge_index, edge_attr, W0, b0, W1, b1, Wg1, bg1, Wg2, bg2,
           gamma1, beta1, gamma2, beta2, Wo, bo):
    pad = jnp.broadcast_to(jnp.array([[0], [TRASH]], jnp.int32),
                           (2, EPAD - E))
    ei = jnp.concatenate([edge_index, pad], axis=1)
    cols3 = ei[1].reshape(NW, NCHUNK, K)
    rc = ei.reshape(2, NW, NCHUNK, K).transpose(1, 2, 0, 3)
    ones128 = jnp.ones((K, H), jnp.float32)
    zerosNH = jnp.zeros((NPAD, H), jnp.float32)

    sc_deg, sc_conv = _sc_kernels()
    degp = sc_deg(cols3, zerosNH, ones128)
    gi, t1, dinv16 = _prologue(x, W0, b0.reshape(1, H), W1, b1.reshape(1, H),
                               Wg1, degp)
    s1p = sc_conv(rc, t1, zerosNH)
    g1, st1 = _combine(s1p, t1, dinv16, bg1.reshape(1, H))
    t2 = _mid(g1, st1, gamma1.reshape(1, H), beta1.reshape(1, H), gi, Wg2,
              dinv16)
    s2p = sc_conv(rc, t2, zerosNH)
    g2, st2 = _combine(s2p, t2, dinv16, bg2.reshape(1, H))
    return _epilogue(g2, st2, gamma2.reshape(1, H), beta2.reshape(1, H),
                     Wo, bo.reshape(1, OUT))


# speedup vs baseline: 2.8263x; 2.8263x over previous
import functools

import jax
import jax.numpy as jnp
from jax import lax
from jax.experimental import pallas as pl
from jax.experimental.pallas import tpu as pltpu
from jax.experimental.pallas import tpu_sc as plsc

N = 10000
E = 320000
D_IN = 26
H = 128
OUT = 64
EPS = 1e-5

NC = 2
NS = 16
NW = NC * NS
EPW = E // NW
K = 128
EPAD = NW * 80 * K
NCHUNK = EPAD // (NW * K)
NPAD = 10240
RPT = NPAD // NS
TRASH = NPAD - 8

B = 1000
GRID = N // B


@functools.cache
def _sc_kernels():
    mesh = plsc.VectorSubcoreMesh(core_axis_name="c", subcore_axis_name="s",
                                  num_cores=NC, num_subcores=NS)

    @functools.partial(
        pl.kernel,
        out_type=jax.ShapeDtypeStruct((NC, NPAD, H), jnp.float32),
        mesh=mesh,
        scratch_types=[
            pltpu.VMEM((NCHUNK, K), jnp.int32),
            pltpu.VMEM((K, H), jnp.float32),
            pltpu.VMEM_SHARED((NPAD, H), jnp.float32),
        ],
    )
    def _sc_deg(cols_hbm, zeros_hbm, ones_hbm, degp_hbm, cidx_v, ones_v, deg_sh):
        c = lax.axis_index("c")
        s = lax.axis_index("s")
        tid = c * NS + s
        pltpu.sync_copy(cols_hbm.at[tid], cidx_v)
        pltpu.sync_copy(ones_hbm, ones_v)
        pltpu.sync_copy(zeros_hbm.at[pl.ds(s * RPT, RPT)],
                        deg_sh.at[pl.ds(s * RPT, RPT)])
        plsc.subcore_barrier()

        def body(j, carry):
            pltpu.sync_copy(ones_v, deg_sh.at[cidx_v.at[j]], add=True)
            return carry

        lax.fori_loop(0, NCHUNK, body, 0)
        plsc.subcore_barrier()
        pltpu.sync_copy(deg_sh.at[pl.ds(s * RPT, RPT)],
                        degp_hbm.at[c, pl.ds(s * RPT, RPT)])

    @functools.partial(
        pl.kernel,
        out_type=jax.ShapeDtypeStruct((NC, NPAD, H), jnp.float32),
        mesh=mesh,
        scratch_types=[
            pltpu.VMEM((2, 2, K), jnp.int32),
            pltpu.VMEM((2, 2, K), jnp.int32),
            pltpu.VMEM((K, H), jnp.float32),
            pltpu.VMEM((K, H), jnp.float32),
            pltpu.VMEM_SHARED((NPAD, H), jnp.float32),
            pltpu.SemaphoreType.DMA,
            pltpu.SemaphoreType.DMA,
            pltpu.SemaphoreType.DMA,
            pltpu.SemaphoreType.DMA,
        ],
    )
    def _sc_conv(rc_hbm, t_hbm, zeros_hbm, sp_hbm,
                 ia_v, ib_v, gbuf0_v, gbuf1_v, acc_sh, sem0, sem1, sema,
                 semb):
        c = lax.axis_index("c")
        s = lax.axis_index("s")
        tid = c * NS + s
        pltpu.sync_copy(zeros_hbm.at[pl.ds(s * RPT, RPT)],
                        acc_sh.at[pl.ds(s * RPT, RPT)])
        plsc.subcore_barrier()

        def g_start(ibuf, row, gb, sem):
            pltpu.async_copy(t_hbm.at[ibuf.at[row, 0]], gb, sem)

        def g_wait(ibuf, row, gb, sem):
            pltpu.make_async_copy(t_hbm.at[ibuf.at[row, 0]], gb, sem).wait()

        def scat(ibuf, row, gb):
            pltpu.sync_copy(gb, acc_sh.at[ibuf.at[row, 1]], add=True)

        def i_start(j2, ibuf, sem):
            pltpu.async_copy(rc_hbm.at[tid, pl.ds(j2, 2)], ibuf, sem)

        def i_wait(j2, ibuf, sem):
            pltpu.make_async_copy(rc_hbm.at[tid, pl.ds(j2, 2)], ibuf,
                                  sem).wait()

        NQUAD = NCHUNK // 4
        pltpu.sync_copy(rc_hbm.at[tid, pl.ds(0, 2)], ia_v)
        i_start(2, ib_v, semb)
        g_start(ia_v, 0, gbuf0_v, sem0)

        def body(i, carry):
            q = 4 * i
            g_start(ia_v, 1, gbuf1_v, sem1)
            g_wait(ia_v, 0, gbuf0_v, sem0)
            scat(ia_v, 0, gbuf0_v)
            i_wait(q + 2, ib_v, semb)
            g_start(ib_v, 0, gbuf0_v, sem0)
            g_wait(ia_v, 1, gbuf1_v, sem1)
            scat(ia_v, 1, gbuf1_v)

            @pl.when(q + 5 < NCHUNK)
            def _():
                i_start(q + 4, ia_v, sema)

            g_start(ib_v, 1, gbuf1_v, sem1)
            g_wait(ib_v, 0, gbuf0_v, sem0)
            scat(ib_v, 0, gbuf0_v)

            @pl.when(q + 5 < NCHUNK)
            def _():
                i_wait(q + 4, ia_v, sema)
                g_start(ia_v, 0, gbuf0_v, sem0)

            g_wait(ib_v, 1, gbuf1_v, sem1)
            scat(ib_v, 1, gbuf1_v)

            @pl.when(q + 7 < NCHUNK)
            def _():
                i_start(q + 6, ib_v, semb)

            return carry

        lax.fori_loop(0, NQUAD, body, 0)
        plsc.subcore_barrier()
        pltpu.sync_copy(acc_sh.at[pl.ds(s * RPT, RPT)],
                        sp_hbm.at[c, pl.ds(s * RPT, RPT)])

    return _sc_deg, _sc_conv


def _dinv_block(degp):
    deg = degp[0, :, 0:1] + degp[1, :, 0:1] + 1.0
    return lax.rsqrt(deg)


def _prologue_body(x_ref, w0_ref, b0_ref, w1_ref, b1_ref, wg1_ref, degp_ref,
                   gi_ref, t1_ref, dinv16_ref):
    h = jnp.maximum(
        jnp.dot(x_ref[...], w0_ref[...], preferred_element_type=jnp.float32)
        + b0_ref[...], 0.0)
    gi = jnp.dot(h, w1_ref[...], preferred_element_type=jnp.float32) + b1_ref[...]
    gi_ref[...] = gi
    dinv = _dinv_block(degp_ref[...])
    dinv16_ref[...] = dinv * jnp.ones((1, 16), jnp.float32)
    t1_ref[...] = jnp.dot(gi, wg1_ref[...],
                          preferred_element_type=jnp.float32) * dinv


def _prologue(x, W0, b0, W1, b1, Wg1, degp):
    return pl.pallas_call(
        _prologue_body,
        grid=(GRID,),
        in_specs=[
            pl.BlockSpec((B, D_IN), lambda i: (i, 0)),
            pl.BlockSpec((D_IN, H), lambda i: (0, 0)),
            pl.BlockSpec((1, H), lambda i: (0, 0)),
            pl.BlockSpec((H, H), lambda i: (0, 0)),
            pl.BlockSpec((1, H), lambda i: (0, 0)),
            pl.BlockSpec((H, H), lambda i: (0, 0)),
            pl.BlockSpec((NC, B, H), lambda i: (0, i, 0)),
        ],
        out_specs=[
            pl.BlockSpec((B, H), lambda i: (i, 0)),
            pl.BlockSpec((B, H), lambda i: (i, 0)),
            pl.BlockSpec((B, 16), lambda i: (i, 0)),
        ],
        out_shape=[
            jax.ShapeDtypeStruct((N, H), jnp.float32),
            jax.ShapeDtypeStruct((N, H), jnp.float32),
            jax.ShapeDtypeStruct((N, 16), jnp.float32),
        ],
    )(x, W0, b0, W1, b1, Wg1, degp)


def _combine_body(sp_ref, t_ref, dinv16_ref, bg_ref, g_ref, st_ref):
    i = pl.program_id(0)
    sp = sp_ref[...]
    dinv = dinv16_ref[...][:, 0:1]
    g = (sp[0] + sp[1] + t_ref[...]) * dinv + bg_ref[...]
    g_ref[...] = g

    @pl.when(i == 0)
    def _():
        st_ref[...] = jnp.zeros((8, H), jnp.float32)

    st_ref[0:1, :] += jnp.sum(g, axis=0, keepdims=True)
    st_ref[1:2, :] += jnp.sum(g * g, axis=0, keepdims=True)


def _combine(sp, t, dinv16, bg):
    return pl.pallas_call(
        _combine_body,
        grid=(GRID,),
        in_specs=[
            pl.BlockSpec((NC, B, H), lambda i: (0, i, 0)),
            pl.BlockSpec((B, H), lambda i: (i, 0)),
            pl.BlockSpec((B, 16), lambda i: (i, 0)),
            pl.BlockSpec((1, H), lambda i: (0, 0)),
        ],
        out_specs=[
            pl.BlockSpec((B, H), lambda i: (i, 0)),
            pl.BlockSpec((8, H), lambda i: (0, 0)),
        ],
        out_shape=[
            jax.ShapeDtypeStruct((N, H), jnp.float32),
            jax.ShapeDtypeStruct((8, H), jnp.float32),
        ],
    )(sp, t, dinv16, bg)


def _mid_body(g_ref, st_ref, gamma_ref, beta_ref, gi_ref, wg2_ref, dinv16_ref,
              t2_ref):
    mu = st_ref[0:1, :] / N
    var = st_ref[1:2, :] / N - mu * mu
    inv = lax.rsqrt(var + EPS)
    xh = (g_ref[...] - mu) * (gamma_ref[...] * inv) + beta_ref[...]
    r = jnp.maximum(xh, 0.0) + gi_ref[...]
    dinv = dinv16_ref[...][:, 0:1]
    t2_ref[...] = jnp.dot(r, wg2_ref[...],
                          preferred_element_type=jnp.float32) * dinv


def _mid(g1, st1, gamma1, beta1, gi, Wg2, dinv16):
    return pl.pallas_call(
        _mid_body,
        grid=(GRID,),
        in_specs=[
            pl.BlockSpec((B, H), lambda i: (i, 0)),
            pl.BlockSpec((8, H), lambda i: (0, 0)),
            pl.BlockSpec((1, H), lambda i: (0, 0)),
            pl.BlockSpec((1, H), lambda i: (0, 0)),
            pl.BlockSpec((B, H), lambda i: (i, 0)),
            pl.BlockSpec((H, H), lambda i: (0, 0)),
            pl.BlockSpec((B, 16), lambda i: (i, 0)),
        ],
        out_specs=pl.BlockSpec((B, H), lambda i: (i, 0)),
        out_shape=jax.ShapeDtypeStruct((N, H), jnp.float32),
    )(g1, st1, gamma1, beta1, gi, Wg2, dinv16)


def _epilogue_body(g_ref, st_ref, gamma_ref, beta_ref, wo_ref, bo_ref, o_ref):
    mu = st_ref[0:1, :] / N
    var = st_ref[1:2, :] / N - mu * mu
    inv = lax.rsqrt(var + EPS)
    xh = (g_ref[...] - mu) * (gamma_ref[...] * inv) + beta_ref[...]
    o_ref[...] = jnp.dot(xh, wo_ref[...],
                         preferred_element_type=jnp.float32) + bo_ref[...]


def _epilogue(g2, st2, gamma2, beta2, Wo, bo):
    return pl.pallas_call(
        _epilogue_body,
        grid=(GRID,),
        in_specs=[
            pl.BlockSpec((B, H), lambda i: (i, 0)),
            pl.BlockSpec((8, H), lambda i: (0, 0)),
            pl.BlockSpec((1, H), lambda i: (0, 0)),
            pl.BlockSpec((1, H), lambda i: (0, 0)),
            pl.BlockSpec((H, OUT), lambda i: (0, 0)),
            pl.BlockSpec((1, OUT), lambda i: (0, 0)),
        ],
        out_specs=pl.BlockSpec((B, OUT), lambda i: (i, 0)),
        out_shape=jax.ShapeDtypeStruct((N, OUT), jnp.float32),
    )(g2, st2, gamma2, beta2, Wo, bo)


def kernel(x, edge_index, edge_attr, W0, b0, W1, b1, Wg1, bg1, Wg2, bg2,
           gamma1, beta1, gamma2, beta2, Wo, bo):
    npads = EPAD - E
    pad_src = (jnp.arange(npads, dtype=jnp.int32) * 61) % N
    pad_dst = N + (jnp.arange(npads, dtype=jnp.int32) % (NPAD - N))
    pad = jnp.stack([pad_src, pad_dst])
    ei = jnp.concatenate([edge_index, pad], axis=1)
    cols3 = ei[1].reshape(NW, NCHUNK, K)
    rc = ei.reshape(2, NW, NCHUNK, K).transpose(1, 2, 0, 3)
    ones128 = jnp.ones((K, H), jnp.float32)
    zerosNH = jnp.zeros((NPAD, H), jnp.float32)

    sc_deg, sc_conv = _sc_kernels()
    degp = sc_deg(cols3, zerosNH, ones128)
    gi, t1, dinv16 = _prologue(x, W0, b0.reshape(1, H), W1, b1.reshape(1, H),
                               Wg1, degp)
    s1p = sc_conv(rc, t1, zerosNH)
    g1, st1 = _combine(s1p, t1, dinv16, bg1.reshape(1, H))
    t2 = _mid(g1, st1, gamma1.reshape(1, H), beta1.reshape(1, H), gi, Wg2,
              dinv16)
    s2p = sc_conv(rc, t2, zerosNH)
    g2, st2 = _combine(s2p, t2, dinv16, bg2.reshape(1, H))
    return _epilogue(g2, st2, gamma2.reshape(1, H), beta2.reshape(1, H),
                     Wo, bo.reshape(1, OUT))
